# baseline (speedup 1.0000x reference)
import functools

import jax
import jax.numpy as jnp
from jax import lax
from jax.experimental import pallas as pl
from jax.experimental.pallas import tpu as pltpu
from jax.experimental.pallas import tpu_sc as plsc

N = 20000
NUM_CLASSES = 80
SCORE_THRESHOLD = 0.5
IOU_THRESHOLD = 0.5

L = 16
NC, NS = 2, 16
NW = NC * NS
P = N + NUM_CLASSES * L
SEG = 96
OUTP = N + L


def _nms_body(packed, segh, vch, keep_out,
              x1v, y1v, x2v, y2v, sxv, kv, segv, vcv, sem):
    wid = lax.axis_index("s") * NC + lax.axis_index("c")

    pltpu.sync_copy(packed.at[pl.ds(0 * P, P)], x1v.at[pl.ds(0, P)])
    pltpu.sync_copy(packed.at[pl.ds(1 * P, P)], y1v.at[pl.ds(0, P)])
    pltpu.sync_copy(packed.at[pl.ds(2 * P, P)], x2v.at[pl.ds(0, P)])
    pltpu.sync_copy(packed.at[pl.ds(3 * P, P)], y2v.at[pl.ds(0, P)])
    pltpu.sync_copy(packed.at[pl.ds(4 * P, P)], sxv.at[pl.ds(0, P)])
    pltpu.sync_copy(segh, segv)
    pltpu.sync_copy(vch, vcv)

    c_lo = (wid * NUM_CLASSES) // NW
    c_hi = ((wid + 1) * NUM_CLASSES) // NW

    def _sload(ref, i):
        return ref[pl.ds(i, L)][0]

    lane = lax.iota(jnp.int32, L)

    def class_body(c, carry):
        start = _sload(segv, c)
        seg_end = _sload(segv, c + 1)
        vcnt = _sload(vcv, c)
        hi_chunk = (vcnt + (L - 1)) // L

        def k_init(jc, carry0):
            loc = jc * L + lane
            kv[pl.ds(start + jc * L, L)] = jnp.where(
                loc < vcnt, 1.0, 0.0).astype(jnp.float32)
            return carry0

        lax.fori_loop(0, (seg_end - start) // L, k_init, 0)

        def i_body(i, carry2):
            gi = start + i
            ki_s = _sload(kv, gi)
            vx1 = jnp.full((L,), _sload(x1v, gi))
            vy1 = jnp.full((L,), _sload(y1v, gi))
            vx2 = jnp.full((L,), _sload(x2v, gi))
            vy2 = jnp.full((L,), _sload(y2v, gi))
            area_i = (vx2 - vx1) * (vy2 - vy1)
            lo_chunk = jnp.where(ki_s > 0.0, (i + 1) // L, hi_chunk)

            def j_body(jc, carry3):
                jb = start + jc * L
                x1j = x1v[pl.ds(jb, L)]
                y1j = y1v[pl.ds(jb, L)]
                x2j = x2v[pl.ds(jb, L)]
                y2j = y2v[pl.ds(jb, L)]
                ix1 = jnp.maximum(vx1, x1j)
                iy1 = jnp.maximum(vy1, y1j)
                ix2 = jnp.minimum(vx2, x2j)
                iy2 = jnp.minimum(vy2, y2j)
                inter = jnp.maximum(ix2 - ix1, 0.0) * jnp.maximum(iy2 - iy1, 0.0)
                area_j = (x2j - x1j) * (y2j - y1j)
                iou = inter / (area_i + area_j - inter + 1e-6)
                jl = (jc * L) + lane
                supp = (iou >= IOU_THRESHOLD) & (jl > i)
                kj = kv[pl.ds(jb, L)]
                kv[pl.ds(jb, L)] = jnp.where(supp, 0.0, kj)
                return carry3

            lax.fori_loop(lo_chunk, hi_chunk, j_body, 0)
            return carry2

        lax.fori_loop(0, vcnt, i_body, 0)
        return carry

    lax.fori_loop(c_lo, c_hi, class_body, 0)

    w_start = _sload(segv, c_lo)
    w_end = _sload(segv, c_hi)
    nch = (w_end - w_start) // L

    def s_body(t, carry):
        off = pl.multiple_of(w_start + t * L, L)
        jdx = sxv[pl.ds(off, L)].astype(jnp.int32)
        pltpu.make_async_copy(kv.at[pl.ds(off, L)], keep_out.at[jdx], sem).start()
        return carry

    lax.fori_loop(0, nch, s_body, 0)

    def d_body(t, carry):
        pltpu.make_async_copy(
            packed.at[pl.ds(0, L)], kv.at[pl.ds(0, L)], sem).wait()
        return carry

    lax.fori_loop(0, nch, d_body, 0)


_sc_nms_cache = []


def _sc_nms(*args):
    if not _sc_nms_cache:
        _sc_nms_cache.append(functools.partial(
            pl.kernel,
            mesh=plsc.VectorSubcoreMesh(core_axis_name="c", subcore_axis_name="s"),
            out_type=jax.ShapeDtypeStruct((OUTP,), jnp.float32),
            scratch_types=[
                pltpu.VMEM((P + L,), jnp.float32),
                pltpu.VMEM((P + L,), jnp.float32),
                pltpu.VMEM((P + L,), jnp.float32),
                pltpu.VMEM((P + L,), jnp.float32),
                pltpu.VMEM((P + L,), jnp.float32),
                pltpu.VMEM((P + L,), jnp.float32),
                pltpu.VMEM((SEG,), jnp.int32),
                pltpu.VMEM((SEG,), jnp.int32),
                pltpu.SemaphoreType.DMA,
            ],
        )(_nms_body))
    return _sc_nms_cache[0](*args)


def kernel(boxes, scores, labels):
    order = jnp.argsort(-scores)
    b = boxes[order]
    s = scores[order]
    l = labels[order]

    order2 = jnp.argsort(l, stable=True)
    b2 = b[order2]
    l2 = l[order2]

    ones = jnp.ones((N,), jnp.int32)
    counts = jnp.zeros((NUM_CLASSES,), jnp.int32).at[l].add(ones)
    valid = (s >= SCORE_THRESHOLD).astype(jnp.int32)
    vcount = jnp.zeros((NUM_CLASSES,), jnp.int32).at[l].add(valid)

    padded = ((counts + (L - 1)) // L) * L
    pstart = jnp.concatenate(
        [jnp.zeros((1,), jnp.int32), jnp.cumsum(padded, dtype=jnp.int32)])
    ustart = jnp.concatenate(
        [jnp.zeros((1,), jnp.int32), jnp.cumsum(counts, dtype=jnp.int32)[:-1]])
    pos = pstart[l2] + (jnp.arange(N, dtype=jnp.int32) - ustart[l2])

    vals = jnp.concatenate(
        [b2.T, order2.astype(jnp.float32)[None, :]], axis=0)
    packed = jnp.full((5, P), 0.0, jnp.float32)
    packed = packed.at[4].set(jnp.float32(N)).at[:, pos].set(vals).reshape(5 * P)
    seg = jnp.zeros((SEG,), jnp.int32).at[:NUM_CLASSES + 1].set(pstart)
    vc = jnp.zeros((SEG,), jnp.int32).at[:NUM_CLASSES].set(vcount)

    keep = _sc_nms(packed, seg, vc)[:N]

    det = jnp.concatenate([b, s[:, None]], axis=1)
    return jnp.where(((keep > 0.0) & (s >= SCORE_THRESHOLD))[:, None], det, 0.0)
